# baseline (speedup 1.0000x reference)
import functools

import jax
import jax.numpy as jnp
from jax.experimental import pallas as pl
from jax.experimental.pallas import tpu as pltpu

_LATENT_DIM = 64
_NUM_CODES = 1024
_ROWS = 8192
_PIECE_ROWS = 4096
_N_PIECES = _ROWS // _PIECE_ROWS
_SUB_ROWS = 256
_SUBS_PER_PIECE = _PIECE_ROWS // _SUB_ROWS
_N_CHUNKS = _ROWS // _SUB_ROWS


def _vq_body(x_hbm, emb_ref, o_hbm, xbuf, obuf, insem, outsem):
    emb = emb_ref[...]
    e2 = jnp.sum(emb * emb, axis=0, keepdims=True)
    half_e2 = 0.5 * e2
    hi = half_e2.astype(jnp.bfloat16).astype(jnp.float32)
    lo = half_e2 - hi
    embaug = jnp.concatenate([emb, -hi, -lo], axis=0)

    def in_copy(p):
        return pltpu.make_async_copy(
            x_hbm.at[pl.ds(p * _PIECE_ROWS, _PIECE_ROWS), :],
            xbuf.at[p % 2], insem.at[p % 2])

    def out_copy(k):
        return pltpu.make_async_copy(
            obuf.at[k],
            o_hbm.at[pl.ds(k * _SUB_ROWS, _SUB_ROWS), :], outsem.at[k])

    in_copy(0).start()
    for p in range(_N_PIECES):
        if p + 1 < _N_PIECES:
            in_copy(p + 1).start()
        in_copy(p).wait()
        for j in range(_SUBS_PER_PIECE):
            k = p * _SUBS_PER_PIECE + j
            xb = xbuf[p % 2, pl.ds(j * _SUB_ROWS, _SUB_ROWS), :]
            xaug = jnp.concatenate(
                [xb, jnp.ones((_SUB_ROWS, 2), jnp.float32)], axis=1)
            scores = jnp.dot(xaug, embaug,
                             preferred_element_type=jnp.float32)
            idx = jnp.argmax(scores, axis=1).reshape(-1, 1)
            col = jax.lax.broadcasted_iota(jnp.int32, scores.shape, 1)
            onehot = (col == idx).astype(jnp.float32)
            obuf[k] = jax.lax.dot_general(
                onehot, emb, (((1,), (1,)), ((), ())),
                preferred_element_type=jnp.float32)
            out_copy(k).start()
    for k in range(_N_CHUNKS):
        out_copy(k).wait()


@functools.partial(jax.jit, static_argnames=("interpret",))
def kernel(x, embeddings, interpret=False):
    orig_shape = x.shape
    xf = x.reshape(-1, _LATENT_DIM)
    out = pl.pallas_call(
        _vq_body,
        in_specs=[
            pl.BlockSpec(memory_space=pltpu.MemorySpace.HBM),
            pl.BlockSpec(memory_space=pltpu.MemorySpace.VMEM),
        ],
        out_specs=pl.BlockSpec(memory_space=pltpu.MemorySpace.HBM),
        out_shape=jax.ShapeDtypeStruct((_ROWS, _LATENT_DIM), jnp.float32),
        scratch_shapes=[
            pltpu.MemorySpace.VMEM((2, _PIECE_ROWS, _LATENT_DIM), jnp.float32),
            pltpu.MemorySpace.VMEM((_N_CHUNKS, _SUB_ROWS, _LATENT_DIM),
                                   jnp.float32),
            pltpu.SemaphoreType.DMA((2,)),
            pltpu.SemaphoreType.DMA((_N_CHUNKS,)),
        ],
        interpret=interpret,
    )(xf, embeddings)
    return out.reshape(orig_shape)


# speedup vs baseline: 1.0013x; 1.0013x over previous
import functools

import jax
import jax.numpy as jnp
from jax.experimental import pallas as pl
from jax.experimental.pallas import tpu as pltpu

_LATENT_DIM = 64
_NUM_CODES = 1024
_BATCH = 8
_SEQ = 1024
_ROWS = _BATCH * _SEQ
_PIECE_BATCH = 4
_PIECE_ROWS = _PIECE_BATCH * _SEQ
_N_PIECES = _BATCH // _PIECE_BATCH
_SUB_ROWS = 256
_SUBS_PER_SEQ = _SEQ // _SUB_ROWS
_N_CHUNKS = _ROWS // _SUB_ROWS


def _vq_body(x_hbm, emb_ref, o_hbm, xbuf, obuf, insem, outsem):
    emb = emb_ref[...]
    e2 = jnp.sum(emb * emb, axis=0, keepdims=True)
    half_e2 = 0.5 * e2
    hi = half_e2.astype(jnp.bfloat16).astype(jnp.float32)
    lo = half_e2 - hi
    embaug = jnp.concatenate([emb, -hi, -lo], axis=0)

    def in_copy(p):
        return pltpu.make_async_copy(
            x_hbm.at[pl.ds(p * _PIECE_BATCH, _PIECE_BATCH), :, :],
            xbuf.at[p % 2], insem.at[p % 2])

    def out_copy(k):
        b, s = divmod(k, _SUBS_PER_SEQ)
        return pltpu.make_async_copy(
            obuf.at[k],
            o_hbm.at[b, pl.ds(s * _SUB_ROWS, _SUB_ROWS), :], outsem.at[k])

    in_copy(0).start()
    for p in range(_N_PIECES):
        if p + 1 < _N_PIECES:
            in_copy(p + 1).start()
        in_copy(p).wait()
        for j in range(_PIECE_ROWS // _SUB_ROWS):
            k = p * (_PIECE_ROWS // _SUB_ROWS) + j
            jb, js = divmod(j, _SUBS_PER_SEQ)
            xb = xbuf[p % 2, jb, pl.ds(js * _SUB_ROWS, _SUB_ROWS), :]
            xaug = jnp.concatenate(
                [xb, jnp.ones((_SUB_ROWS, 2), jnp.float32)], axis=1)
            scores = jnp.dot(xaug, embaug,
                             preferred_element_type=jnp.float32)
            idx = jnp.argmax(scores, axis=1).reshape(-1, 1)
            col = jax.lax.broadcasted_iota(jnp.int32, scores.shape, 1)
            onehot = (col == idx).astype(jnp.float32)
            obuf[k] = jax.lax.dot_general(
                onehot, emb, (((1,), (1,)), ((), ())),
                preferred_element_type=jnp.float32)
            out_copy(k).start()
    for k in range(_N_CHUNKS):
        out_copy(k).wait()


@functools.partial(jax.jit, static_argnames=("interpret",))
def kernel(x, embeddings, interpret=False):
    return pl.pallas_call(
        _vq_body,
        in_specs=[
            pl.BlockSpec(memory_space=pltpu.MemorySpace.HBM),
            pl.BlockSpec(memory_space=pltpu.MemorySpace.VMEM),
        ],
        out_specs=pl.BlockSpec(memory_space=pltpu.MemorySpace.HBM),
        out_shape=jax.ShapeDtypeStruct((_BATCH, _SEQ, _LATENT_DIM),
                                       jnp.float32),
        scratch_shapes=[
            pltpu.MemorySpace.VMEM(
                (2, _PIECE_BATCH, _SEQ, _LATENT_DIM), jnp.float32),
            pltpu.MemorySpace.VMEM((_N_CHUNKS, _SUB_ROWS, _LATENT_DIM),
                                   jnp.float32),
            pltpu.SemaphoreType.DMA((2,)),
            pltpu.SemaphoreType.DMA((_N_CHUNKS,)),
        ],
        interpret=interpret,
    )(x, embeddings)
